# Initial kernel scaffold; baseline (speedup 1.0000x reference)
import jax
import jax.numpy as jnp
from jax import lax
from jax.experimental import pallas as pl
from jax.experimental.pallas import tpu as pltpu
from jax.experimental.pallas import tpu_sc as plsc

N = 10000
D = 128
E = 640000

NP = 10240
NR = NP // 128
NC, NS = 2, 16
NW = NC * NS
CH = 128
CPW = 157
EP = NW * CPW * CH
RPW = NP // NS

_mesh = plsc.VectorSubcoreMesh(core_axis_name="c", subcore_axis_name="s")


def _worker(cid, sid):
    return sid * NC + cid


def _zero_acc(acc_v):
    zero16 = jnp.zeros((16,), jnp.float32)

    @pl.loop(0, NR)
    def _z(i):
        for j in range(8):
            acc_v[i, pl.ds(j * 16, 16)] = zero16


def _split_rc(idx):
    return lax.shift_right_logical(idx, 7), lax.bitwise_and(idx, 127)


def _sc_deg_body(e_ref, out_ref, idx_v, acc_v):
    w = _worker(lax.axis_index("c"), lax.axis_index("s"))
    _zero_acc(acc_v)
    pltpu.sync_copy(e_ref.at[1, pl.ds(w * CPW, CPW)], idx_v)
    ones16 = jnp.ones((16,), jnp.float32)

    @pl.loop(0, CPW)
    def _edges(g):
        for j in range(8):
            r, c = _split_rc(idx_v[g, pl.ds(j * 16, 16)])
            plsc.addupdate_scatter(acc_v, [r, c], ones16)

    pltpu.sync_copy(acc_v, out_ref.at[w])


_sc_deg = pl.kernel(
    _sc_deg_body,
    out_type=jax.ShapeDtypeStruct((NW, NR, 128), jnp.float32),
    mesh=_mesh,
    scratch_types=[
        pltpu.VMEM((CPW, CH), jnp.int32),
        pltpu.VMEM((NR, 128), jnp.float32),
    ],
)


def _sc_agg_body(u_ref, e_ref, z_ref, out_ref, src_v, dst_v, rows_v, acc_sh, sem):
    cid = lax.axis_index("c")
    sid = lax.axis_index("s")
    w = _worker(cid, sid)
    pltpu.sync_copy(z_ref, acc_sh.at[pl.ds(sid * RPW, RPW)])
    pltpu.sync_copy(e_ref.at[0, pl.ds(w * CPW, CPW)], src_v)
    pltpu.sync_copy(e_ref.at[1, pl.ds(w * CPW, CPW)], dst_v)
    plsc.subcore_barrier()

    @pl.loop(0, CPW)
    def _edges(g):
        pltpu.async_copy(u_ref.at[src_v.at[g]], rows_v, sem).wait()
        pltpu.sync_copy(rows_v, acc_sh.at[dst_v.at[g]], add=True)

    plsc.subcore_barrier()
    pltpu.sync_copy(
        acc_sh.at[pl.ds(sid * RPW, RPW)], out_ref.at[cid, pl.ds(sid * RPW, RPW)]
    )


_sc_agg = pl.kernel(
    _sc_agg_body,
    out_type=jax.ShapeDtypeStruct((NC, NP, 128), jnp.float32),
    mesh=_mesh,
    scratch_types=[
        pltpu.VMEM((CPW, CH), jnp.int32),
        pltpu.VMEM((CPW, CH), jnp.int32),
        pltpu.VMEM((CH, 128), jnp.float32),
        pltpu.VMEM_SHARED((NP, 128), jnp.float32),
        pltpu.SemaphoreType.DMA,
    ],
)


def _sc_sagg_body(t_ref, e_ref, out_ref, src_v, dst_v, t_v, acc_v):
    w = _worker(lax.axis_index("c"), lax.axis_index("s"))
    pltpu.sync_copy(t_ref, t_v)
    pltpu.sync_copy(e_ref.at[0, pl.ds(w * CPW, CPW)], src_v)
    pltpu.sync_copy(e_ref.at[1, pl.ds(w * CPW, CPW)], dst_v)
    _zero_acc(acc_v)

    @pl.loop(0, CPW)
    def _edges(g):
        for j in range(8):
            rs, cs = _split_rc(src_v[g, pl.ds(j * 16, 16)])
            vals = plsc.load_gather(t_v, [rs, cs])
            rd, cd = _split_rc(dst_v[g, pl.ds(j * 16, 16)])
            plsc.addupdate_scatter(acc_v, [rd, cd], vals)

    pltpu.sync_copy(acc_v, out_ref.at[w])


_sc_sagg = pl.kernel(
    _sc_sagg_body,
    out_type=jax.ShapeDtypeStruct((NW, NR, 128), jnp.float32),
    mesh=_mesh,
    scratch_types=[
        pltpu.VMEM((CPW, CH), jnp.int32),
        pltpu.VMEM((CPW, CH), jnp.int32),
        pltpu.VMEM((NR, 128), jnp.float32),
        pltpu.VMEM((NR, 128), jnp.float32),
    ],
)


BM = 1024


def _tc_dis_body(degp_ref, dis_ref):
    deg = jnp.sum(degp_ref[...], axis=0) + 1.0
    dis_ref[...] = lax.rsqrt(deg)


_tc_dis = pl.pallas_call(
    _tc_dis_body,
    out_shape=jax.ShapeDtypeStruct((NR, 128), jnp.float32),
)


def _tc_u_body(x_ref, w1_ref, dis_ref, u_ref):
    h = jnp.dot(x_ref[...], w1_ref[...], preferred_element_type=jnp.float32)
    u_ref[...] = dis_ref[...] * h


_tc_u = pl.pallas_call(
    _tc_u_body,
    grid=(NP // BM,),
    in_specs=[
        pl.BlockSpec((BM, D), lambda i: (i, 0)),
        pl.BlockSpec((D, D), lambda i: (0, 0)),
        pl.BlockSpec((BM, 1), lambda i: (i, 0)),
    ],
    out_specs=pl.BlockSpec((BM, D), lambda i: (i, 0)),
    out_shape=jax.ShapeDtypeStruct((NP, D), jnp.float32),
)


def _tc_t_body(aggp_ref, u_ref, dis_ref, b1_ref, w2_ref, t_ref):
    agg = aggp_ref[0] + aggp_ref[1]
    out1 = dis_ref[...] * (agg + u_ref[...]) + b1_ref[...]
    r = jnp.maximum(out1, 0.0)
    s = jnp.sum(r * w2_ref[...], axis=1, keepdims=True)
    t_ref[...] = dis_ref[...] * s


_tc_t = pl.pallas_call(
    _tc_t_body,
    grid=(NP // BM,),
    in_specs=[
        pl.BlockSpec((NC, BM, D), lambda i: (0, i, 0)),
        pl.BlockSpec((BM, D), lambda i: (i, 0)),
        pl.BlockSpec((BM, 1), lambda i: (i, 0)),
        pl.BlockSpec((1, D), lambda i: (0, 0)),
        pl.BlockSpec((1, D), lambda i: (0, 0)),
    ],
    out_specs=pl.BlockSpec((BM, 1), lambda i: (i, 0)),
    out_shape=jax.ShapeDtypeStruct((NP, 1), jnp.float32),
)


def _tc_out_body(qp_ref, t_ref, dis_ref, b2_ref, o_ref):
    q = jnp.sum(qp_ref[...], axis=0)
    o_ref[...] = dis_ref[...] * (q + t_ref[...]) + b2_ref[0, 0]


_tc_out = pl.pallas_call(
    _tc_out_body,
    out_shape=jax.ShapeDtypeStruct((NR, 128), jnp.float32),
)


def kernel(x, edge_index, W1, b1, W2, b2):
    xp = jnp.zeros((NP, D), jnp.float32).at[:N].set(x)
    ep = jnp.pad(edge_index, ((0, 0), (0, EP - E)), constant_values=NP - 1)
    e3 = ep.reshape(2, EP // CH, CH)

    degp = _sc_deg(e3)
    dis80 = _tc_dis(degp)
    dis_col = dis80.reshape(NP, 1)
    u = _tc_u(xp, W1, dis_col)
    zeros = jnp.zeros((RPW, D), jnp.float32)
    aggp = _sc_agg(u, e3, zeros)
    t_col = _tc_t(aggp, u, dis_col, b1.reshape(1, D), W2.reshape(1, D))
    t80 = t_col.reshape(NR, 128)
    qp = _sc_sagg(t80, e3)
    o80 = _tc_out(qp, t80, dis80, b2.reshape(1, 1))
    return o80.reshape(NP, 1)[:N]


# speedup vs baseline: 17.0132x; 17.0132x over previous
import jax
import jax.numpy as jnp
from jax import lax
from jax.experimental import pallas as pl
from jax.experimental.pallas import tpu as pltpu
from jax.experimental.pallas import tpu_sc as plsc

N = 10000
D = 128
E = 640000

NP = 10240
NR = NP // 128
NC, NS = 2, 16
NW = NC * NS
CH = 128
CPW = 160
EP = NW * CPW * CH
RPW = NP // NS

_mesh = plsc.VectorSubcoreMesh(core_axis_name="c", subcore_axis_name="s")
_sc_params = pltpu.CompilerParams(needs_layout_passes=False)


def _worker(cid, sid):
    return sid * NC + cid


def _zero_acc(acc_v):
    zero16 = jnp.zeros((16,), jnp.float32)

    @pl.loop(0, NR)
    def _z(i):
        for j in range(8):
            acc_v[i, pl.ds(j * 16, 16)] = zero16


def _split_rc(idx):
    return lax.shift_right_logical(idx, 7), lax.bitwise_and(idx, 127)


def _sc_deg_body(e_ref, out_ref, idx_v, acc_v):
    w = _worker(lax.axis_index("c"), lax.axis_index("s"))
    _zero_acc(acc_v)
    pltpu.sync_copy(e_ref.at[1, pl.ds(w * CPW, CPW)], idx_v)
    ones16 = jnp.ones((16,), jnp.float32)

    @pl.loop(0, CPW)
    def _edges(g):
        for j in range(8):
            r, c = _split_rc(idx_v[g, pl.ds(j * 16, 16)])
            plsc.addupdate_scatter(acc_v, [r, c], ones16)

    pltpu.sync_copy(acc_v, out_ref.at[w])


_sc_deg = pl.kernel(
    _sc_deg_body,
    out_type=jax.ShapeDtypeStruct((NW, NR, 128), jnp.float32),
    mesh=_mesh,
    compiler_params=_sc_params,
    scratch_types=[
        pltpu.VMEM((CPW, CH), jnp.int32),
        pltpu.VMEM((NR, 128), jnp.float32),
    ],
)


GRP = 8
NG = CPW // GRP


def _sc_agg_body(u_ref, e_ref, z_ref, out_ref, src_v, dst_v, rows_v, acc_sh, sem):
    cid = lax.axis_index("c")
    sid = lax.axis_index("s")
    w = _worker(cid, sid)
    pltpu.sync_copy(z_ref, acc_sh.at[pl.ds(sid * RPW, RPW)])
    plsc.subcore_barrier()

    @pl.loop(0, NG)
    def _grp(gg):
        base = w * CPW + gg * GRP
        pltpu.sync_copy(e_ref.at[0, pl.ds(base, GRP)], src_v)
        pltpu.sync_copy(e_ref.at[1, pl.ds(base, GRP)], dst_v)
        for j in range(GRP):
            pltpu.async_copy(u_ref.at[src_v.at[j]], rows_v, sem).wait()
            pltpu.sync_copy(rows_v, acc_sh.at[dst_v.at[j]], add=True)

    plsc.subcore_barrier()
    pltpu.sync_copy(
        acc_sh.at[pl.ds(sid * RPW, RPW)], out_ref.at[cid, pl.ds(sid * RPW, RPW)]
    )


_sc_agg = pl.kernel(
    _sc_agg_body,
    out_type=jax.ShapeDtypeStruct((NC, NP, 128), jnp.float32),
    mesh=_mesh,
    compiler_params=_sc_params,
    scratch_types=[
        pltpu.VMEM((GRP, CH), jnp.int32),
        pltpu.VMEM((GRP, CH), jnp.int32),
        pltpu.VMEM((CH, 128), jnp.float32),
        pltpu.VMEM_SHARED((NP, 128), jnp.float32),
        pltpu.SemaphoreType.DMA,
    ],
)


def _sc_sagg_body(t_ref, e_ref, out_ref, src_v, dst_v, t_v, acc_v):
    w = _worker(lax.axis_index("c"), lax.axis_index("s"))
    pltpu.sync_copy(t_ref, t_v)
    pltpu.sync_copy(e_ref.at[0, pl.ds(w * CPW, CPW)], src_v)
    pltpu.sync_copy(e_ref.at[1, pl.ds(w * CPW, CPW)], dst_v)
    _zero_acc(acc_v)

    @pl.loop(0, CPW)
    def _edges(g):
        for j in range(8):
            rs, cs = _split_rc(src_v[g, pl.ds(j * 16, 16)])
            vals = plsc.load_gather(t_v, [rs, cs])
            rd, cd = _split_rc(dst_v[g, pl.ds(j * 16, 16)])
            plsc.addupdate_scatter(acc_v, [rd, cd], vals)

    pltpu.sync_copy(acc_v, out_ref.at[w])


_sc_sagg = pl.kernel(
    _sc_sagg_body,
    out_type=jax.ShapeDtypeStruct((NW, NR, 128), jnp.float32),
    mesh=_mesh,
    compiler_params=_sc_params,
    scratch_types=[
        pltpu.VMEM((CPW, CH), jnp.int32),
        pltpu.VMEM((CPW, CH), jnp.int32),
        pltpu.VMEM((NR, 128), jnp.float32),
        pltpu.VMEM((NR, 128), jnp.float32),
    ],
)


BM = 1024


def _tc_dis_body(degp_ref, dis_ref):
    deg = jnp.sum(degp_ref[...], axis=0) + 1.0
    dis_ref[...] = lax.rsqrt(deg)


_tc_dis = pl.pallas_call(
    _tc_dis_body,
    out_shape=jax.ShapeDtypeStruct((NR, 128), jnp.float32),
)


def _tc_u_body(x_ref, w1_ref, dis_ref, u_ref):
    h = jnp.dot(x_ref[...], w1_ref[...], preferred_element_type=jnp.float32)
    u_ref[...] = dis_ref[...] * h


_tc_u = pl.pallas_call(
    _tc_u_body,
    grid=(NP // BM,),
    in_specs=[
        pl.BlockSpec((BM, D), lambda i: (i, 0)),
        pl.BlockSpec((D, D), lambda i: (0, 0)),
        pl.BlockSpec((BM, 1), lambda i: (i, 0)),
    ],
    out_specs=pl.BlockSpec((BM, D), lambda i: (i, 0)),
    out_shape=jax.ShapeDtypeStruct((NP, D), jnp.float32),
)


def _tc_t_body(aggp_ref, u_ref, dis_ref, b1_ref, w2_ref, t_ref):
    agg = aggp_ref[0] + aggp_ref[1]
    out1 = dis_ref[...] * (agg + u_ref[...]) + b1_ref[...]
    r = jnp.maximum(out1, 0.0)
    s = jnp.sum(r * w2_ref[...], axis=1, keepdims=True)
    t_ref[...] = dis_ref[...] * s


_tc_t = pl.pallas_call(
    _tc_t_body,
    grid=(NP // BM,),
    in_specs=[
        pl.BlockSpec((NC, BM, D), lambda i: (0, i, 0)),
        pl.BlockSpec((BM, D), lambda i: (i, 0)),
        pl.BlockSpec((BM, 1), lambda i: (i, 0)),
        pl.BlockSpec((1, D), lambda i: (0, 0)),
        pl.BlockSpec((1, D), lambda i: (0, 0)),
    ],
    out_specs=pl.BlockSpec((BM, 1), lambda i: (i, 0)),
    out_shape=jax.ShapeDtypeStruct((NP, 1), jnp.float32),
)


def _tc_out_body(qp_ref, t_ref, dis_ref, b2_ref, o_ref):
    q = jnp.sum(qp_ref[...], axis=0)
    o_ref[...] = dis_ref[...] * (q + t_ref[...]) + b2_ref[0, 0]


_tc_out = pl.pallas_call(
    _tc_out_body,
    out_shape=jax.ShapeDtypeStruct((NR, 128), jnp.float32),
)


def kernel(x, edge_index, W1, b1, W2, b2):
    xp = jnp.zeros((NP, D), jnp.float32).at[:N].set(x)
    ep = jnp.pad(edge_index, ((0, 0), (0, EP - E)), constant_values=NP - 1)
    e3 = ep.reshape(2, EP // CH, CH)

    degp = _sc_deg(e3)
    dis80 = _tc_dis(degp)
    dis_col = dis80.reshape(NP, 1)
    u = _tc_u(xp, W1, dis_col)
    zeros = jnp.zeros((RPW, D), jnp.float32)
    aggp = _sc_agg(u, e3, zeros)
    t_col = _tc_t(aggp, u, dis_col, b1.reshape(1, D), W2.reshape(1, D))
    t80 = t_col.reshape(NR, 128)
    qp = _sc_sagg(t80, e3)
    o80 = _tc_out(qp, t80, dis80, b2.reshape(1, 1))
    return o80.reshape(NP, 1)[:N]


# speedup vs baseline: 29.6333x; 1.7418x over previous
import jax
import jax.numpy as jnp
from jax import lax
from jax.experimental import pallas as pl
from jax.experimental.pallas import tpu as pltpu
from jax.experimental.pallas import tpu_sc as plsc

N = 10000
D = 128
E = 640000

NP = 10240
NR = NP // 128
NC, NS = 2, 16
NW = NC * NS
CH = 128
CPW = 160
EP = NW * CPW * CH
RPW = NP // NS

_mesh = plsc.VectorSubcoreMesh(core_axis_name="c", subcore_axis_name="s")
_sc_params = pltpu.CompilerParams(needs_layout_passes=False)
_sc_params_lin = pltpu.CompilerParams(
    needs_layout_passes=False, use_tc_tiling_on_sc=False
)


def _worker(cid, sid):
    return sid * NC + cid


def _zero_acc(acc_v):
    zero16 = jnp.zeros((16,), jnp.float32)

    @pl.loop(0, NR)
    def _z(i):
        for j in range(8):
            acc_v[i, pl.ds(j * 16, 16)] = zero16


def _split_rc(idx):
    return lax.shift_right_logical(idx, 7), lax.bitwise_and(idx, 127)


def _sc_deg_body(e_ref, out_ref, idx_v, acc_v):
    w = _worker(lax.axis_index("c"), lax.axis_index("s"))
    _zero_acc(acc_v)
    pltpu.sync_copy(e_ref.at[1, pl.ds(w * CPW, CPW)], idx_v)
    ones16 = jnp.ones((16,), jnp.float32)

    @pl.loop(0, CPW)
    def _edges(g):
        for j in range(8):
            r, c = _split_rc(idx_v[g, pl.ds(j * 16, 16)])
            plsc.addupdate_scatter(acc_v, [r, c], ones16)

    pltpu.sync_copy(acc_v, out_ref.at[w])


_sc_deg = pl.kernel(
    _sc_deg_body,
    out_type=jax.ShapeDtypeStruct((NW, NR, 128), jnp.float32),
    mesh=_mesh,
    compiler_params=_sc_params,
    scratch_types=[
        pltpu.VMEM((CPW, CH), jnp.int32),
        pltpu.VMEM((NR, 128), jnp.float32),
    ],
)


FH = 64
CPC = EP // CH
CPT = CPC // NS
GRP = 16
NGT = CPT // GRP
NB = 8
DELTA = 4


def _sc_agg_body(u_ref, e_ref, z_ref, out_ref, src_v, dst_v, *bufs):
    rows = bufs[:NB]
    sg = bufs[NB:2 * NB]
    ss = bufs[2 * NB:3 * NB]
    cid = lax.axis_index("c")
    sid = lax.axis_index("s")
    acc_sh = bufs[3 * NB]
    pltpu.sync_copy(z_ref, acc_sh.at[pl.ds(sid * RPW, RPW)])
    plsc.subcore_barrier()

    def stage(gg, par):
        base = sid * CPT + gg * GRP
        pltpu.sync_copy(e_ref.at[0, pl.ds(base, GRP)], src_v.at[par])
        pltpu.sync_copy(e_ref.at[1, pl.ds(base, GRP)], dst_v.at[par])

    def gather_start(par, j, rb):
        pltpu.async_copy(u_ref.at[cid].at[src_v.at[par, j]], rows[rb], sg[rb])

    def gather_wait(rb):
        pltpu.make_async_copy(u_ref.at[cid, pl.ds(0, CH)], rows[rb], sg[rb]).wait()

    def scatter_start(par, j, rb):
        pltpu.async_copy(rows[rb], acc_sh.at[dst_v.at[par, j]], ss[rb], add=True)

    def scatter_wait(rb):
        pltpu.make_async_copy(rows[rb], acc_sh.at[pl.ds(0, CH)], ss[rb]).wait()

    stage(0, 0)
    for j in range(GRP):
        rb = j % NB
        if j >= NB:
            scatter_wait(rb)
        gather_start(0, j, rb)
        if j >= DELTA:
            gather_wait((j - DELTA) % NB)
            scatter_start(0, j - DELTA, (j - DELTA) % NB)

    @pl.loop(1, NGT)
    def _grp(gg):
        par = lax.bitwise_and(gg, 1)
        stage(gg, par)
        for j in range(GRP):
            rb = j % NB
            scatter_wait(rb)
            gather_start(par, j, rb)
            rb2 = (j - DELTA) % NB
            gather_wait(rb2)
            if j >= DELTA:
                scatter_start(par, j - DELTA, rb2)
            else:
                scatter_start(1 - par, j + GRP - DELTA, rb2)

    for j in range(GRP - DELTA, GRP):
        rb2 = j % NB
        gather_wait(rb2)
        scatter_start((NGT - 1) % 2, j, rb2)
    for rb in range(NB):
        scatter_wait(rb)

    plsc.subcore_barrier()
    pltpu.sync_copy(
        acc_sh.at[pl.ds(sid * RPW, RPW)], out_ref.at[cid, pl.ds(sid * RPW, RPW)]
    )


_sc_agg = pl.kernel(
    _sc_agg_body,
    out_type=jax.ShapeDtypeStruct((NC, NP, FH), jnp.float32),
    mesh=_mesh,
    compiler_params=_sc_params_lin,
    scratch_types=[
        pltpu.VMEM((2, GRP, CH), jnp.int32),
        pltpu.VMEM((2, GRP, CH), jnp.int32),
        *[pltpu.VMEM((CH, FH), jnp.float32) for _ in range(NB)],
        *[pltpu.SemaphoreType.DMA for _ in range(2 * NB)],
        pltpu.VMEM_SHARED((NP, FH), jnp.float32),
    ],
)


def _sc_sagg_body(t_ref, e_ref, out_ref, src_v, dst_v, t_v, acc_v):
    w = _worker(lax.axis_index("c"), lax.axis_index("s"))
    pltpu.sync_copy(t_ref, t_v)
    pltpu.sync_copy(e_ref.at[0, pl.ds(w * CPW, CPW)], src_v)
    pltpu.sync_copy(e_ref.at[1, pl.ds(w * CPW, CPW)], dst_v)
    _zero_acc(acc_v)

    @pl.loop(0, CPW)
    def _edges(g):
        for j in range(8):
            rs, cs = _split_rc(src_v[g, pl.ds(j * 16, 16)])
            vals = plsc.load_gather(t_v, [rs, cs])
            rd, cd = _split_rc(dst_v[g, pl.ds(j * 16, 16)])
            plsc.addupdate_scatter(acc_v, [rd, cd], vals)

    pltpu.sync_copy(acc_v, out_ref.at[w])


_sc_sagg = pl.kernel(
    _sc_sagg_body,
    out_type=jax.ShapeDtypeStruct((NW, NR, 128), jnp.float32),
    mesh=_mesh,
    compiler_params=_sc_params,
    scratch_types=[
        pltpu.VMEM((CPW, CH), jnp.int32),
        pltpu.VMEM((CPW, CH), jnp.int32),
        pltpu.VMEM((NR, 128), jnp.float32),
        pltpu.VMEM((NR, 128), jnp.float32),
    ],
)


BM = 1024


def _tc_dis_body(degp_ref, dis_ref):
    deg = jnp.sum(degp_ref[...], axis=0) + 1.0
    dis_ref[...] = lax.rsqrt(deg)


_tc_dis = pl.pallas_call(
    _tc_dis_body,
    out_shape=jax.ShapeDtypeStruct((NR, 128), jnp.float32),
)


def _tc_u_body(x_ref, w1_ref, dis_ref, u_ref):
    h = jnp.dot(x_ref[...], w1_ref[...], preferred_element_type=jnp.float32)
    u = dis_ref[...] * h
    u_ref[0] = u[:, :FH]
    u_ref[1] = u[:, FH:]


_tc_u = pl.pallas_call(
    _tc_u_body,
    grid=(NP // BM,),
    in_specs=[
        pl.BlockSpec((BM, D), lambda i: (i, 0)),
        pl.BlockSpec((D, D), lambda i: (0, 0)),
        pl.BlockSpec((BM, 1), lambda i: (i, 0)),
    ],
    out_specs=pl.BlockSpec((NC, BM, FH), lambda i: (0, i, 0)),
    out_shape=jax.ShapeDtypeStruct((NC, NP, FH), jnp.float32),
)


def _tc_t_body(aggp_ref, u_ref, dis_ref, b1_ref, w2_ref, t_ref):
    dis = dis_ref[...]
    b1 = b1_ref[...]
    w2 = w2_ref[...]
    s = jnp.zeros((BM, 1), jnp.float32)
    for k in range(NC):
        o = dis * (aggp_ref[k] + u_ref[k]) + b1[k:k + 1]
        r = jnp.maximum(o, 0.0)
        s = s + jnp.sum(r * w2[k:k + 1], axis=1, keepdims=True)
    t_ref[...] = dis * s


_tc_t = pl.pallas_call(
    _tc_t_body,
    grid=(NP // BM,),
    in_specs=[
        pl.BlockSpec((NC, BM, FH), lambda i: (0, i, 0)),
        pl.BlockSpec((NC, BM, FH), lambda i: (0, i, 0)),
        pl.BlockSpec((BM, 1), lambda i: (i, 0)),
        pl.BlockSpec((NC, FH), lambda i: (0, 0)),
        pl.BlockSpec((NC, FH), lambda i: (0, 0)),
    ],
    out_specs=pl.BlockSpec((BM, 1), lambda i: (i, 0)),
    out_shape=jax.ShapeDtypeStruct((NP, 1), jnp.float32),
)


def _tc_out_body(qp_ref, t_ref, dis_ref, b2_ref, o_ref):
    q = jnp.sum(qp_ref[...], axis=0)
    o_ref[...] = dis_ref[...] * (q + t_ref[...]) + b2_ref[0, 0]


_tc_out = pl.pallas_call(
    _tc_out_body,
    out_shape=jax.ShapeDtypeStruct((NR, 128), jnp.float32),
)


def kernel(x, edge_index, W1, b1, W2, b2):
    xp = jnp.zeros((NP, D), jnp.float32).at[:N].set(x)
    ep = jnp.pad(edge_index, ((0, 0), (0, EP - E)), constant_values=NP - 1)
    e3 = ep.reshape(2, EP // CH, CH)

    degp = _sc_deg(e3)
    dis80 = _tc_dis(degp)
    dis_col = dis80.reshape(NP, 1)
    u2 = _tc_u(xp, W1, dis_col)
    zeros = jnp.zeros((RPW, FH), jnp.float32)
    aggp = _sc_agg(u2, e3, zeros)
    t_col = _tc_t(aggp, u2, dis_col, b1.reshape(NC, FH), W2.reshape(NC, FH))
    t80 = t_col.reshape(NR, 128)
    qp = _sc_sagg(t80, e3)
    o80 = _tc_out(qp, t80, dis80, b2.reshape(1, 1))
    return o80.reshape(NP, 1)[:N]


# speedup vs baseline: 54.0000x; 1.8223x over previous
import jax
import jax.numpy as jnp
from jax import lax
from jax.experimental import pallas as pl
from jax.experimental.pallas import tpu as pltpu
from jax.experimental.pallas import tpu_sc as plsc

N = 10000
D = 128
E = 640000

NP = 10240
NR = NP // 128
NC, NS = 2, 16
NW = NC * NS
CH = 128
CPW = 160
EP = NW * CPW * CH
RPW = NP // NS

_mesh = plsc.VectorSubcoreMesh(core_axis_name="c", subcore_axis_name="s")
_sc_params = pltpu.CompilerParams(needs_layout_passes=False)
_sc_params_lin = pltpu.CompilerParams(
    needs_layout_passes=False, use_tc_tiling_on_sc=False
)


def _worker(cid, sid):
    return sid * NC + cid


def _zero_acc(acc_v):
    zero16 = jnp.zeros((16,), jnp.float32)

    @pl.loop(0, NR)
    def _z(i):
        for j in range(8):
            acc_v[i, pl.ds(j * 16, 16)] = zero16


def _split_rc(idx):
    return lax.shift_right_logical(idx, 7), lax.bitwise_and(idx, 127)


def _sc_deg_body(e_ref, out_ref, idx_v, acc_v):
    w = _worker(lax.axis_index("c"), lax.axis_index("s"))
    _zero_acc(acc_v)
    pltpu.sync_copy(e_ref.at[1, pl.ds(w * CPW, CPW)], idx_v)
    ones16 = jnp.ones((16,), jnp.float32)

    @pl.loop(0, CPW)
    def _edges(g):
        for j in range(8):
            r, c = _split_rc(idx_v[g, pl.ds(j * 16, 16)])
            plsc.addupdate_scatter(acc_v, [r, c], ones16)

    pltpu.sync_copy(acc_v, out_ref.at[w])


_sc_deg = pl.kernel(
    _sc_deg_body,
    out_type=jax.ShapeDtypeStruct((NW, NR, 128), jnp.float32),
    mesh=_mesh,
    compiler_params=_sc_params,
    scratch_types=[
        pltpu.VMEM((CPW, CH), jnp.int32),
        pltpu.VMEM((NR, 128), jnp.float32),
    ],
)


FH = 64
CPC = EP // CH
CPT = CPC // NS
GRP = 16
NGT = CPT // GRP
NB = 4
DELTA = 2


def _sc_agg_body(u_ref, e_ref, z_ref, out_ref, src_v, dst_v, *bufs):
    rows = bufs[:NB]
    sg = bufs[NB:2 * NB]
    ss = bufs[2 * NB:3 * NB]
    cid = lax.axis_index("c")
    sid = lax.axis_index("s")
    acc_sh = bufs[3 * NB]
    u_sh = bufs[3 * NB + 1]
    pltpu.sync_copy(z_ref, acc_sh.at[pl.ds(sid * RPW, RPW)])
    pltpu.sync_copy(u_ref.at[cid, pl.ds(sid * RPW, RPW)], u_sh.at[pl.ds(sid * RPW, RPW)])
    plsc.subcore_barrier()

    def stage(gg, par):
        base = sid * CPT + gg * GRP
        pltpu.sync_copy(e_ref.at[0, pl.ds(base, GRP)], src_v.at[par])
        pltpu.sync_copy(e_ref.at[1, pl.ds(base, GRP)], dst_v.at[par])

    def gather_start(par, j, rb):
        pltpu.async_copy(u_sh.at[src_v.at[par, j]], rows[rb], sg[rb])

    def gather_wait(rb):
        pltpu.make_async_copy(u_ref.at[cid, pl.ds(0, CH)], rows[rb], sg[rb]).wait()

    def scatter_start(par, j, rb):
        pltpu.async_copy(rows[rb], acc_sh.at[dst_v.at[par, j]], ss[rb], add=True)

    def scatter_wait(rb):
        pltpu.make_async_copy(rows[rb], acc_sh.at[pl.ds(0, CH)], ss[rb]).wait()

    stage(0, 0)
    for j in range(GRP):
        rb = j % NB
        if j >= NB:
            scatter_wait(rb)
        gather_start(0, j, rb)
        if j >= DELTA:
            gather_wait((j - DELTA) % NB)
            scatter_start(0, j - DELTA, (j - DELTA) % NB)

    @pl.loop(1, NGT)
    def _grp(gg):
        par = lax.bitwise_and(gg, 1)
        stage(gg, par)
        for j in range(GRP):
            rb = j % NB
            scatter_wait(rb)
            gather_start(par, j, rb)
            rb2 = (j - DELTA) % NB
            gather_wait(rb2)
            if j >= DELTA:
                scatter_start(par, j - DELTA, rb2)
            else:
                scatter_start(1 - par, j + GRP - DELTA, rb2)

    for j in range(GRP - DELTA, GRP):
        rb2 = j % NB
        gather_wait(rb2)
        scatter_start((NGT - 1) % 2, j, rb2)
    for rb in range(NB):
        scatter_wait(rb)

    plsc.subcore_barrier()
    pltpu.sync_copy(
        acc_sh.at[pl.ds(sid * RPW, RPW)], out_ref.at[cid, pl.ds(sid * RPW, RPW)]
    )


_sc_agg = pl.kernel(
    _sc_agg_body,
    out_type=jax.ShapeDtypeStruct((NC, NP, FH), jnp.float32),
    mesh=_mesh,
    compiler_params=_sc_params_lin,
    scratch_types=[
        pltpu.VMEM((2, GRP, CH), jnp.int32),
        pltpu.VMEM((2, GRP, CH), jnp.int32),
        *[pltpu.VMEM((CH, FH), jnp.float32) for _ in range(NB)],
        *[pltpu.SemaphoreType.DMA for _ in range(2 * NB)],
        pltpu.VMEM_SHARED((NP, FH), jnp.float32),
        pltpu.VMEM_SHARED((NP, FH), jnp.float32),
    ],
)


def _sc_sagg_body(t_ref, e_ref, out_ref, src_v, dst_v, t_v, acc_v):
    w = _worker(lax.axis_index("c"), lax.axis_index("s"))
    pltpu.sync_copy(t_ref, t_v)
    pltpu.sync_copy(e_ref.at[0, pl.ds(w * CPW, CPW)], src_v)
    pltpu.sync_copy(e_ref.at[1, pl.ds(w * CPW, CPW)], dst_v)
    _zero_acc(acc_v)

    @pl.loop(0, CPW)
    def _edges(g):
        for j in range(8):
            rs, cs = _split_rc(src_v[g, pl.ds(j * 16, 16)])
            vals = plsc.load_gather(t_v, [rs, cs])
            rd, cd = _split_rc(dst_v[g, pl.ds(j * 16, 16)])
            plsc.addupdate_scatter(acc_v, [rd, cd], vals)

    pltpu.sync_copy(acc_v, out_ref.at[w])


_sc_sagg = pl.kernel(
    _sc_sagg_body,
    out_type=jax.ShapeDtypeStruct((NW, NR, 128), jnp.float32),
    mesh=_mesh,
    compiler_params=_sc_params,
    scratch_types=[
        pltpu.VMEM((CPW, CH), jnp.int32),
        pltpu.VMEM((CPW, CH), jnp.int32),
        pltpu.VMEM((NR, 128), jnp.float32),
        pltpu.VMEM((NR, 128), jnp.float32),
    ],
)


BM = 1024


def _tc_dis_body(degp_ref, dis_ref):
    deg = jnp.sum(degp_ref[...], axis=0) + 1.0
    dis_ref[...] = lax.rsqrt(deg)


_tc_dis = pl.pallas_call(
    _tc_dis_body,
    out_shape=jax.ShapeDtypeStruct((NR, 128), jnp.float32),
)


def _tc_u_body(x_ref, w1_ref, dis_ref, u_ref):
    h = jnp.dot(x_ref[...], w1_ref[...], preferred_element_type=jnp.float32)
    u = dis_ref[...] * h
    u_ref[0] = u[:, :FH]
    u_ref[1] = u[:, FH:]


_tc_u = pl.pallas_call(
    _tc_u_body,
    grid=(NP // BM,),
    in_specs=[
        pl.BlockSpec((BM, D), lambda i: (i, 0)),
        pl.BlockSpec((D, D), lambda i: (0, 0)),
        pl.BlockSpec((BM, 1), lambda i: (i, 0)),
    ],
    out_specs=pl.BlockSpec((NC, BM, FH), lambda i: (0, i, 0)),
    out_shape=jax.ShapeDtypeStruct((NC, NP, FH), jnp.float32),
)


def _tc_t_body(aggp_ref, u_ref, dis_ref, b1_ref, w2_ref, t_ref):
    dis = dis_ref[...]
    b1 = b1_ref[...]
    w2 = w2_ref[...]
    s = jnp.zeros((BM, 1), jnp.float32)
    for k in range(NC):
        o = dis * (aggp_ref[k] + u_ref[k]) + b1[k:k + 1]
        r = jnp.maximum(o, 0.0)
        s = s + jnp.sum(r * w2[k:k + 1], axis=1, keepdims=True)
    t_ref[...] = dis * s


_tc_t = pl.pallas_call(
    _tc_t_body,
    grid=(NP // BM,),
    in_specs=[
        pl.BlockSpec((NC, BM, FH), lambda i: (0, i, 0)),
        pl.BlockSpec((NC, BM, FH), lambda i: (0, i, 0)),
        pl.BlockSpec((BM, 1), lambda i: (i, 0)),
        pl.BlockSpec((NC, FH), lambda i: (0, 0)),
        pl.BlockSpec((NC, FH), lambda i: (0, 0)),
    ],
    out_specs=pl.BlockSpec((BM, 1), lambda i: (i, 0)),
    out_shape=jax.ShapeDtypeStruct((NP, 1), jnp.float32),
)


def _tc_out_body(qp_ref, t_ref, dis_ref, b2_ref, o_ref):
    q = jnp.sum(qp_ref[...], axis=0)
    o_ref[...] = dis_ref[...] * (q + t_ref[...]) + b2_ref[0, 0]


_tc_out = pl.pallas_call(
    _tc_out_body,
    out_shape=jax.ShapeDtypeStruct((NR, 128), jnp.float32),
)


def kernel(x, edge_index, W1, b1, W2, b2):
    xp = jnp.zeros((NP, D), jnp.float32).at[:N].set(x)
    ep = jnp.pad(edge_index, ((0, 0), (0, EP - E)), constant_values=NP - 1)
    e3 = ep.reshape(2, EP // CH, CH)

    degp = _sc_deg(e3)
    dis80 = _tc_dis(degp)
    dis_col = dis80.reshape(NP, 1)
    u2 = _tc_u(xp, W1, dis_col)
    zeros = jnp.zeros((RPW, FH), jnp.float32)
    aggp = _sc_agg(u2, e3, zeros)
    t_col = _tc_t(aggp, u2, dis_col, b1.reshape(NC, FH), W2.reshape(NC, FH))
    t80 = t_col.reshape(NR, 128)
    qp = _sc_sagg(t80, e3)
    o80 = _tc_out(qp, t80, dis80, b2.reshape(1, 1))
    return o80.reshape(NP, 1)[:N]
